# baseline (speedup 1.0000x reference)
import functools

import jax
import jax.numpy as jnp
from jax import lax
from jax.experimental import pallas as pl
from jax.experimental.pallas import tpu as pltpu
from jax.experimental.pallas import tpu_sc as plsc

LANES = 16
LANE = 128
N_WORKERS = 32
CHUNK_ROWS = 8
N_SLOTS = 3


def _make_sc_kernel(batch, seq, dims):
    rows_per_worker = seq // N_WORKERS
    n_chunks = rows_per_worker // CHUNK_ROWS
    mesh = plsc.VectorSubcoreMesh(core_axis_name="c", subcore_axis_name="s")

    @functools.partial(
        pl.kernel,
        out_type=jax.ShapeDtypeStruct((batch, seq, dims), jnp.float32),
        mesh=mesh,
        compiler_params=pltpu.CompilerParams(use_tc_tiling_on_sc=True),
        scratch_types=(
            [pltpu.VMEM((batch, CHUNK_ROWS, dims), jnp.float32) for _ in range(N_SLOTS)]
            + [pltpu.VMEM((CHUNK_ROWS, dims), jnp.float32) for _ in range(N_SLOTS)]
            + [pltpu.SemaphoreType.DMA] * (2 * N_SLOTS)
        ),
    )
    def sc_add(tok_hbm, pos_hbm, out_hbm, *rest):
        tokbuf = rest[:N_SLOTS]
        posbuf = rest[N_SLOTS:2 * N_SLOTS]
        ld = rest[2 * N_SLOTS:3 * N_SLOTS]
        st = rest[3 * N_SLOTS:4 * N_SLOTS]
        wid = lax.axis_index("s") * 2 + lax.axis_index("c")
        base = wid * rows_per_worker

        def tok_copy(k, row):
            return pltpu.make_async_copy(
                tok_hbm.at[:, pl.ds(row, CHUNK_ROWS), :], tokbuf[k], ld[k]
            )

        def pos_copy(k, row):
            return pltpu.make_async_copy(
                pos_hbm.at[pl.ds(row, CHUNK_ROWS), :], posbuf[k], ld[k]
            )

        def store_copy(k, row):
            return pltpu.make_async_copy(
                tokbuf[k], out_hbm.at[:, pl.ds(row, CHUNK_ROWS), :], st[k]
            )

        def fire_loads(k, row):
            tok_copy(k, row).start()
            pos_copy(k, row).start()

        def compute(k):
            def body(r, carry):
                for c0 in range(0, dims, LANES):
                    s = pl.ds(c0, LANES)
                    pv = posbuf[k][r, s]
                    for b in range(batch):
                        tokbuf[k][b, r, s] = tokbuf[k][b, r, s] + pv
                return carry

            lax.fori_loop(0, CHUNK_ROWS, body, 0)

        def chunk_body(c, carry):
            for k in range(N_SLOTS):
                @pl.when(c % N_SLOTS == k)
                def _(k=k):
                    k_next = (k + 1) % N_SLOTS
                    row = base + c * CHUNK_ROWS

                    @pl.when(c + 1 < n_chunks)
                    def _():
                        @pl.when(c >= N_SLOTS - 1)
                        def _():
                            store_copy(k_next, base).wait()

                        fire_loads(k_next, row + CHUNK_ROWS)

                    tok_copy(k, row).wait()
                    pos_copy(k, row).wait()
                    compute(k)
                    store_copy(k, row).start()
            return carry

        fire_loads(0, base)
        lax.fori_loop(0, n_chunks, chunk_body, 0)

        for c in range(max(0, n_chunks - N_SLOTS), n_chunks):
            store_copy(c % N_SLOTS, base).wait()

    return sc_add


def kernel(token_embeddings, pos_table):
    batch, seq, dims = token_embeddings.shape
    sc_add = _make_sc_kernel(batch, seq, dims)
    return sc_add(token_embeddings, pos_table)
